# baseline (speedup 1.0000x reference)
import functools

import jax
import jax.numpy as jnp
from jax import lax
from jax.experimental import pallas as pl
from jax.experimental.pallas import tpu as pltpu
from jax.experimental.pallas import tpu_sc as plsc

_DIM = 128
_WORDS = _DIM // 2
_GROUPS = _DIM // 32
_NC = 2
_NS = 16
_NW = _NC * _NS
_CHUNK = 128


@functools.lru_cache(maxsize=None)
def _make_gather(batch, seq):
    n_rows = batch * seq
    rows_per_w = n_rows // _NW
    n_chunks = rows_per_w // _CHUNK
    w_per_b = seq // rows_per_w
    mesh = plsc.VectorSubcoreMesh(core_axis_name="c", subcore_axis_name="s")
    n_buf = 4

    @functools.partial(
        pl.kernel,
        mesh=mesh,
        compiler_params=pltpu.CompilerParams(
            needs_layout_passes=False, use_tc_tiling_on_sc=False),
        out_type=[
            jax.ShapeDtypeStruct((batch, seq, 1, _DIM), jnp.float32),
            jax.ShapeDtypeStruct((batch, seq, 1, _DIM), jnp.float32),
        ],
        scratch_types=(
            [pltpu.VMEM((rows_per_w,), jnp.int32)]
            + [pltpu.VMEM((_CHUNK, _WORDS), jnp.int32)] * n_buf
            + [pltpu.VMEM((_CHUNK, _DIM), jnp.float32)] * n_buf
            + [pltpu.SemaphoreType.DMA] * (2 * n_buf)
        ),
    )
    def gather_kernel(cos_hbm, sin_hbm, idx_hbm, cos_out, sin_out,
                      idx_v, *bufs_and_sems):
        gbufs = bufs_and_sems[:n_buf]
        fbufs = bufs_and_sems[n_buf:2 * n_buf]
        gsems = bufs_and_sems[2 * n_buf:3 * n_buf]
        wsems = bufs_and_sems[3 * n_buf:]
        wid = lax.axis_index("s") * _NC + lax.axis_index("c")
        b = wid // w_per_b
        s0 = (wid % w_per_b) * rows_per_w
        pltpu.sync_copy(idx_hbm.at[b, pl.ds(s0, rows_per_w)], idx_v)

        steps = []
        for table, out in ((cos_hbm, cos_out), (sin_hbm, sin_out)):
            for c in range(n_chunks):
                steps.append((table, out, c))
        n = len(steps)

        ghandles = [None] * n
        whandles = [None] * n

        def start_gather(i):
            table, _, c = steps[i]
            ghandles[i] = pltpu.async_copy(
                table.at[idx_v.at[pl.ds(c * _CHUNK, _CHUNK)]],
                gbufs[i % n_buf], gsems[i % n_buf])

        def convert(gbuf, fbuf):
            @pl.loop(0, _CHUNK, unroll=2)
            def _row(r):
                for g in range(_GROUPS):
                    w = gbuf[r, pl.ds(g * 16, 16)]
                    lo = plsc.bitcast(w << 16, jnp.float32)
                    hi = plsc.bitcast(w & jnp.int32(-65536), jnp.float32)
                    fbuf[r, pl.ds(g * 32, 16)] = lo
                    fbuf[r, pl.ds(g * 32 + 16, 16)] = hi

        for i in range(min(n_buf, n)):
            start_gather(i)
        for i in range(n):
            _, out, c = steps[i]
            ghandles[i].wait()
            if i >= n_buf:
                whandles[i - n_buf].wait()
            convert(gbufs[i % n_buf], fbufs[i % n_buf])
            whandles[i] = pltpu.async_copy(
                fbufs[i % n_buf],
                out.at[b, pl.ds(s0 + c * _CHUNK, _CHUNK), 0],
                wsems[i % n_buf])
            if i + n_buf < n:
                start_gather(i + n_buf)
        for i in range(max(0, n - n_buf), n):
            whandles[i].wait()

    return gather_kernel


def _pack_pairs(table):
    v = table.shape[0]
    z = table.reshape(v, _GROUPS, 2, 16).swapaxes(2, 3)
    return jax.lax.bitcast_convert_type(z, jnp.int32).reshape(v, _WORDS)


def kernel(x, pos_ids, cos_cached, sin_cached):
    b, s = pos_ids.shape
    cos_t = _pack_pairs(cos_cached)
    sin_t = _pack_pairs(sin_cached)
    idx = pos_ids.astype(jnp.int32)
    cos_r, sin_r = _make_gather(b, s)(cos_t, sin_t, idx)
    return (cos_r, sin_r)


# speedup vs baseline: 1.2538x; 1.2538x over previous
import functools

import jax
import jax.numpy as jnp
from jax import lax
from jax.experimental import pallas as pl
from jax.experimental.pallas import tpu as pltpu
from jax.experimental.pallas import tpu_sc as plsc

_DIM = 128
_WORDS = _DIM // 2
_GROUPS = _DIM // 32
_NC = 2
_NS = 16
_NW = _NC * _NS
_CHUNK = 128


@functools.lru_cache(maxsize=None)
def _make_gather(batch, seq):
    n_rows = batch * seq
    rows_per_w = n_rows // _NW
    n_chunks = rows_per_w // _CHUNK
    w_per_b = seq // rows_per_w
    mesh = plsc.VectorSubcoreMesh(core_axis_name="c", subcore_axis_name="s")
    n_buf = 4

    @functools.partial(
        pl.kernel,
        mesh=mesh,
        compiler_params=pltpu.CompilerParams(
            needs_layout_passes=False, use_tc_tiling_on_sc=False),
        out_type=[
            jax.ShapeDtypeStruct((batch, seq, 1, _DIM), jnp.float32),
            jax.ShapeDtypeStruct((batch, seq, 1, _DIM), jnp.float32),
        ],
        scratch_types=(
            [pltpu.VMEM((rows_per_w,), jnp.int32)]
            + [pltpu.VMEM((_CHUNK, _WORDS), jnp.int32)] * n_buf
            + [pltpu.VMEM((_CHUNK, _DIM), jnp.float32)] * n_buf
            + [pltpu.SemaphoreType.DMA] * (2 * n_buf)
        ),
    )
    def gather_kernel(cos_hbm, sin_hbm, idx_hbm, cos_out, sin_out,
                      idx_v, *bufs_and_sems):
        gbufs = bufs_and_sems[:n_buf]
        fbufs = bufs_and_sems[n_buf:2 * n_buf]
        gsems = bufs_and_sems[2 * n_buf:3 * n_buf]
        wsems = bufs_and_sems[3 * n_buf:]
        wid = lax.axis_index("s") * _NC + lax.axis_index("c")
        b = wid // w_per_b
        s0 = (wid % w_per_b) * rows_per_w
        pltpu.sync_copy(idx_hbm.at[b, pl.ds(s0, rows_per_w)], idx_v)

        steps = []
        for table, out in ((cos_hbm, cos_out), (sin_hbm, sin_out)):
            for c in range(n_chunks):
                steps.append((table, out, c))
        n = len(steps)

        ghandles = [None] * n
        whandles = [None] * n

        def start_gather(i):
            table, _, c = steps[i]
            ghandles[i] = pltpu.async_copy(
                table.at[idx_v.at[pl.ds(c * _CHUNK, _CHUNK)]],
                gbufs[i % n_buf], gsems[i % n_buf])

        def convert(gbuf, fbuf):
            @plsc.parallel_loop(0, _CHUNK, unroll=4)
            def _row(r):
                for g in range(_GROUPS):
                    w = gbuf[r, pl.ds(g * 16, 16)]
                    lo = plsc.bitcast(w << 16, jnp.float32)
                    hi = plsc.bitcast(w & jnp.int32(-65536), jnp.float32)
                    fbuf[r, pl.ds(g * 32, 16)] = lo
                    fbuf[r, pl.ds(g * 32 + 16, 16)] = hi

        for i in range(min(n_buf, n)):
            start_gather(i)
        for i in range(n):
            _, out, c = steps[i]
            ghandles[i].wait()
            if i >= n_buf:
                whandles[i - n_buf].wait()
            convert(gbufs[i % n_buf], fbufs[i % n_buf])
            whandles[i] = pltpu.async_copy(
                fbufs[i % n_buf],
                out.at[b, pl.ds(s0 + c * _CHUNK, _CHUNK), 0],
                wsems[i % n_buf])
            if i + n_buf < n:
                start_gather(i + n_buf)
        for i in range(max(0, n - n_buf), n):
            whandles[i].wait()

    return gather_kernel


def _pack_pairs(table):
    v = table.shape[0]
    u = jax.lax.bitcast_convert_type(table, jnp.uint16).astype(jnp.int32)
    z = u.reshape(v, _GROUPS, 2, 16)
    w = z[:, :, 0, :] | (z[:, :, 1, :] << 16)
    return w.reshape(v, _WORDS)


def kernel(x, pos_ids, cos_cached, sin_cached):
    b, s = pos_ids.shape
    cos_t = _pack_pairs(cos_cached)
    sin_t = _pack_pairs(sin_cached)
    idx = pos_ids.astype(jnp.int32)
    cos_r, sin_r = _make_gather(b, s)(cos_t, sin_t, idx)
    return (cos_r, sin_r)
